# baseline (speedup 1.0000x reference)
import functools
import jax
import jax.numpy as jnp
from jax import lax
from jax.experimental import pallas as pl
from jax.experimental.pallas import tpu as pltpu
from jax.experimental.pallas import tpu_sc as plsc

_NC, _NS, _L = 2, 16, 16
_NW = _NC * _NS
_BLK = 1000


def _lin_body(x_ref, w_ref, b_ref, o_ref, *, slope):
    acc = jnp.dot(x_ref[...], w_ref[...], preferred_element_type=jnp.float32)
    acc = acc + b_ref[...]
    if slope is not None:
        acc = jnp.where(acc > 0, acc, slope * acc)
    o_ref[...] = acc


def _linear(x, W, b, slope=None, blk=_BLK):
    M, K = x.shape
    N = W.shape[1]
    return pl.pallas_call(
        functools.partial(_lin_body, slope=slope),
        grid=(M // blk,),
        in_specs=[
            pl.BlockSpec((blk, K), lambda i: (i, 0)),
            pl.BlockSpec((K, N), lambda i: (0, 0)),
            pl.BlockSpec((1, N), lambda i: (0, 0)),
        ],
        out_specs=pl.BlockSpec((blk, N), lambda i: (i, 0)),
        out_shape=jax.ShapeDtypeStruct((M, N), jnp.float32),
    )(x, W, b.reshape(1, N))


def _gat_mm_body(x_ref, w_ref, a2_ref, hs_ref, aa_ref):
    acc = jnp.dot(x_ref[...], w_ref[...], preferred_element_type=jnp.float32)
    hs_ref[...] = acc
    aa_ref[...] = jnp.dot(acc, a2_ref[...], preferred_element_type=jnp.float32)


def _gat_mm(x, W, a_src, a_dst, blk=_BLK):
    M, K = x.shape
    N = W.shape[1]
    A2 = jnp.stack([a_src, a_dst], axis=1)
    hs, aa = pl.pallas_call(
        _gat_mm_body,
        grid=(M // blk,),
        in_specs=[
            pl.BlockSpec((blk, K), lambda i: (i, 0)),
            pl.BlockSpec((K, N), lambda i: (0, 0)),
            pl.BlockSpec((N, 2), lambda i: (0, 0)),
        ],
        out_specs=[
            pl.BlockSpec((blk, N), lambda i: (i, 0)),
            pl.BlockSpec((blk, 2), lambda i: (i, 0)),
        ],
        out_shape=[
            jax.ShapeDtypeStruct((M, N), jnp.float32),
            jax.ShapeDtypeStruct((M, 2), jnp.float32),
        ],
    )(x, W, A2)
    return hs, aa[:, 0], aa[:, 1]


def _comb_body(*refs, two, slope):
    if two:
        (numA, sA, hs, aa, mA, biasA, numB, sB, biasB, o_ref) = refs
    else:
        (numA, sA, hs, aa, mA, biasA, o_ref) = refs
    aav = aa[...]
    ps = aav[:, 0:1] + aav[:, 1:2]
    ps = jnp.where(ps > 0, ps, 0.2 * ps)
    ps = jnp.exp(ps - mA[0, 0])
    nA = numA[0] + numA[1] + ps * hs[...]
    dA = sA[0] + sA[1] + ps + 1e-16
    o = nA / dA + biasA[...]
    if two:
        nB = numB[0] + numB[1]
        dB = sB[0] + sB[1] + 1e-16
        o = o + nB / dB + biasB[...]
    o_ref[...] = jnp.where(o > 0, o, slope * o)


def _combine(numA, sA, hs, as_A, ad_A, biasA, extraB, slope, blk=_BLK):
    M, HC = hs.shape
    npad = numA.shape[1]
    mA = jnp.maximum(jnp.max(as_A) + jnp.max(ad_A), 0.0).reshape(1, 1)
    aa = jnp.stack([as_A, ad_A], axis=1)
    two = extraB is not None
    ins = [numA, sA.reshape(_NC, npad, 1), hs, aa, mA, biasA.reshape(1, HC)]
    specs = [
        pl.BlockSpec((_NC, blk, HC), lambda i: (0, i, 0)),
        pl.BlockSpec((_NC, blk, 1), lambda i: (0, i, 0)),
        pl.BlockSpec((blk, HC), lambda i: (i, 0)),
        pl.BlockSpec((blk, 2), lambda i: (i, 0)),
        pl.BlockSpec((1, 1), lambda i: (0, 0)),
        pl.BlockSpec((1, HC), lambda i: (0, 0)),
    ]
    if two:
        numB, sB, biasB = extraB
        ins += [numB, sB.reshape(_NC, npad, 1), biasB.reshape(1, HC)]
        specs += [
            pl.BlockSpec((_NC, blk, HC), lambda i: (0, i, 0)),
            pl.BlockSpec((_NC, blk, 1), lambda i: (0, i, 0)),
            pl.BlockSpec((1, HC), lambda i: (0, 0)),
        ]
    return pl.pallas_call(
        functools.partial(_comb_body, two=two, slope=slope),
        grid=(M // blk,),
        in_specs=specs,
        out_specs=pl.BlockSpec((blk, HC), lambda i: (i, 0)),
        out_shape=jax.ShapeDtypeStruct((M, HC), jnp.float32),
    )(*ins)


def _edge_phase(src, dst, as_v, ad_v, hs, rng):
    E = src.shape[0]
    N, HC = hs.shape
    SL = 128
    nslab = HC // SL
    hs_parts = [hs[:, c * SL:(c + 1) * SL] for c in range(nslab)]
    assert E % _NW == 0
    ew = E // _NW
    assert ew % 8 == 0 and N % _L == 0
    nit = (ew + _L - 1) // _L
    nrounds = -(-N // rng)
    while (nrounds * rng) % (128 * _NS) != 0:
        nrounds += 1
    assert nrounds <= _L
    npad = nrounds * rng
    cap = max(nit * _L, -(-ew // 128) * 128)
    alloc = cap + _L
    nch = cap // 128
    rsub = rng // _NS
    nsub = npad // _NS
    G = 64 if nslab > 2 else 128

    def iota():
        return lax.iota(jnp.int32, _L)

    def body(src_h, dst_h, asv_h, adv_h, *rest):
        hs_hs = rest[:nslab]
        (z2_h, z1_h, num_h, s_h,
         asv, adv, srcb, dstb, p_orig, src_l, dloc_l, p_l,
         gidx0, sidx0, schunk_i, schunk_p) = rest[nslab:nslab + 16]
        k = nslab + 16
        rows0 = rest[k:k + nslab]
        accs = rest[k + nslab:k + 2 * nslab]
        (s_acc, sem, sem_g0, sem_s0) = rest[k + 2 * nslab:]
        c = lax.axis_index("c")
        s_ = lax.axis_index("s")
        wid = c * _NS + s_

        pltpu.sync_copy(asv_h, asv)
        pltpu.sync_copy(adv_h, adv)
        pltpu.sync_copy(src_h.at[pl.ds(wid * ew, ew)], srcb.at[pl.ds(0, ew)])
        pltpu.sync_copy(dst_h.at[pl.ds(wid * ew, ew)], dstb.at[pl.ds(0, ew)])

        zi = jnp.zeros((_L,), jnp.int32)
        zf = jnp.zeros((_L,), jnp.float32)
        j = ew
        while j < alloc - _L + 1:
            plsc.store_scatter(srcb, [j + iota()], zi)
            plsc.store_scatter(dstb, [j + iota()], zi)
            plsc.store_scatter(p_orig, [j + iota()], zf)
            plsc.store_scatter(src_l, [j + iota()], zi)
            plsc.store_scatter(dloc_l, [j + iota()], zi)
            j += _L

        def mbody(ref):
            def f(i, m):
                return jnp.maximum(m, plsc.load_gather(ref, [i * _L + iota()]))
            return jnp.max(lax.fori_loop(0, N // _L, f,
                                         jnp.full((_L,), -3e38, jnp.float32)))
        M = jnp.maximum(mbody(asv) + mbody(adv), 0.0)

        def cbody(i, car):
            pos = i * _L + iota()
            lm = pos < ew
            d16 = plsc.load_gather(dstb, [pos])
            rid = jnp.zeros((_L,), jnp.int32)
            for r in range(1, nrounds):
                rid = rid + (d16 >= r * rng).astype(jnp.int32)
            out = []
            for r in range(nrounds):
                m = (rid == r) & lm
                out.append(car[r] + plsc.all_reduce_population_count(m))
            return tuple(out)
        cnt = lax.fori_loop(0, nit, cbody,
                            tuple(jnp.zeros((_L,), jnp.int32)
                                  for _ in range(nrounds)))
        offv = []
        o = jnp.zeros((_L,), jnp.int32)
        for r in range(nrounds):
            offv.append(o)
            o = o + cnt[r]

        def pbody(i, car):
            pos = i * _L + iota()
            lm = pos < ew
            s16 = plsc.load_gather(srcb, [pos])
            d16 = plsc.load_gather(dstb, [pos])
            a = plsc.load_gather(asv, [s16]) + plsc.load_gather(adv, [d16])
            a = jnp.where(a > 0, a, 0.2 * a)
            p = jnp.exp(a - M)
            p = jnp.where(lm, p, 0.0)
            plsc.store_scatter(p_orig, [pos], p)
            rid = jnp.zeros((_L,), jnp.int32)
            for r in range(1, nrounds):
                rid = rid + (d16 >= r * rng).astype(jnp.int32)
            dloc = d16 - rid * rng
            out = []
            for r in range(nrounds):
                m = (rid == r) & lm
                cs = plsc.cumsum(m.astype(jnp.int32))
                posr = jnp.maximum(offv[r] + car[r] + cs - 1, 0)
                plsc.store_scatter(src_l, [posr], s16, mask=m)
                plsc.store_scatter(dloc_l, [posr], dloc, mask=m)
                plsc.store_scatter(p_l, [posr], p, mask=m)
                out.append(car[r] + plsc.all_reduce_population_count(m))
            return tuple(out)
        lax.fori_loop(0, nit, pbody,
                      tuple(jnp.zeros((_L,), jnp.int32) for _ in range(nrounds)))

        n_vec = jnp.zeros((_L,), jnp.int32)
        off_vec = jnp.zeros((_L,), jnp.int32)
        for r in range(nrounds):
            lane = (iota() == r).astype(jnp.int32)
            n_vec = n_vec + cnt[r] * lane
            off_vec = off_vec + offv[r] * lane

        pltpu.sync_copy(z1_h, s_acc.at[pl.ds(s_ * nsub, nsub)])
        plsc.subcore_barrier()

        def sbody(ci, _):
            for j4 in range(8):
                ix = ci * 128 + j4 * _L + iota()
                schunk_i[pl.ds(j4 * _L, _L)] = plsc.load_gather(dstb, [ix])
                schunk_p[pl.ds(j4 * _L, _L)] = plsc.load_gather(p_orig, [ix])
            pltpu.sync_copy(schunk_p, s_acc.at[schunk_i], add=True)
            return 0
        lax.fori_loop(0, nch, sbody, 0)
        plsc.subcore_barrier()
        pltpu.sync_copy(s_acc.at[pl.ds(s_ * nsub, nsub)],
                        s_h.at[pl.ds(c * npad + s_ * nsub, nsub)])
        plsc.subcore_barrier()

        def round_body(r, _r):
            for cc in range(nslab):
                pltpu.sync_copy(z2_h.at[pl.ds(0, rsub)],
                                accs[cc].at[pl.ds(s_ * rsub, rsub)])
            plsc.subcore_barrier()

            lane = iota() == r
            n_r = jnp.max(jnp.where(lane, n_vec, 0))
            off_r = jnp.max(jnp.where(lane, off_vec, 0))
            buf = [(gidx0, sidx0, rows0, sem_g0, sem_s0)]

            def stage(base, gi, si):
                start = jnp.minimum(off_r + base, alloc - G)
                for j4 in range(G // _L):
                    ix = start + j4 * _L + iota()
                    gi[pl.ds(j4 * _L, _L)] = plsc.load_gather(src_l, [ix])
                    si[pl.ds(j4 * _L, _L)] = plsc.load_gather(dloc_l, [ix])

            def fire_g(which):
                gi, _si, rws, sg, _ss = buf[which]
                for cc in range(nslab):
                    pltpu.async_copy(hs_hs[cc].at[gi], rws[cc], sg)

            def wait_g(which):
                gi, _si, rws, sg, _ss = buf[which]
                for cc in range(nslab):
                    pltpu.make_async_copy(hs_hs[cc].at[gi], rws[cc], sg).wait()

            def fire_s(which):
                _gi, si, rws, _sg, ss = buf[which]
                for cc in range(nslab):
                    pltpu.async_copy(rws[cc], accs[cc].at[si], ss, add=True)

            def wait_s(which):
                _gi, si, rws, _sg, ss = buf[which]
                for cc in range(nslab):
                    pltpu.make_async_copy(rws[cc], accs[cc].at[si], ss).wait()

            def scale(which, base):
                _gi, _si, rws, _sg, _ss = buf[which]

                def gb(g, _2):
                    for j in range(_L):
                        ro = g * _L + j
                        pv = plsc.load_gather(
                            p_l, [jnp.broadcast_to(off_r + base + ro, (_L,))])
                        valid = (base + ro < n_r).astype(jnp.float32)
                        pv = pv * valid
                        ridx = jnp.broadcast_to(ro, (_L,))
                        for cc in range(nslab):
                            for c4 in range(SL // _L):
                                cidx = c4 * _L + iota()
                                v = plsc.load_gather(rws[cc], [ridx, cidx])
                                plsc.store_scatter(rws[cc], [ridx, cidx],
                                                   v * pv)
                    return 0
                lax.fori_loop(0, G // _L, gb, 0)

            def bbody(b, _):
                base = b * G
                stage(base, gidx0, sidx0)
                fire_g(0)
                wait_g(0)
                scale(0, base)
                fire_s(0)
                wait_s(0)
                return 0
            nb = lax.shift_right_logical(n_r + (G - 1), G.bit_length() - 1)
            lax.fori_loop(0, nb, bbody, 0)
            plsc.subcore_barrier()

            for cc in range(nslab):
                pltpu.sync_copy(
                    accs[cc].at[pl.ds(s_ * rsub, rsub)],
                    num_h.at[pl.ds(c * npad + r * rng + s_ * rsub, rsub),
                             pl.ds(cc * SL, SL)])
            plsc.subcore_barrier()
            return 0
        lax.fori_loop(0, nrounds, round_body, 0)

    z2 = jnp.zeros((max(rsub, G), SL), jnp.float32)
    z1 = jnp.zeros((nsub,), jnp.float32)
    mesh = plsc.VectorSubcoreMesh(core_axis_name="c", subcore_axis_name="s")
    f = pl.kernel(
        body,
        out_type=[
            jax.ShapeDtypeStruct((_NC * npad, HC), jnp.float32),
            jax.ShapeDtypeStruct((_NC * npad,), jnp.float32),
        ],
        mesh=mesh,
        compiler_params=pltpu.CompilerParams(needs_layout_passes=False),
        scratch_types=[
            pltpu.VMEM((N,), jnp.float32),
            pltpu.VMEM((N,), jnp.float32),
            pltpu.VMEM((alloc,), jnp.int32),
            pltpu.VMEM((alloc,), jnp.int32),
            pltpu.VMEM((alloc,), jnp.float32),
            pltpu.VMEM((alloc,), jnp.int32),
            pltpu.VMEM((alloc,), jnp.int32),
            pltpu.VMEM((alloc,), jnp.float32),
            pltpu.VMEM((G,), jnp.int32),
            pltpu.VMEM((G,), jnp.int32),
            pltpu.VMEM((128,), jnp.int32),
            pltpu.VMEM((128,), jnp.float32),
        ] + [pltpu.VMEM((G, SL), jnp.float32) for _ in range(nslab)]
        + [pltpu.VMEM_SHARED((rng, SL), jnp.float32) for _ in range(nslab)]
        + [
            pltpu.VMEM_SHARED((npad,), jnp.float32),
            pltpu.SemaphoreType.DMA,
            pltpu.SemaphoreType.DMA,
            pltpu.SemaphoreType.DMA,
        ],
    )
    num, s = f(src, dst, as_v, ad_v, *hs_parts, z2, z1)
    return num.reshape(_NC, npad, HC), s.reshape(_NC, npad)


def kernel(x_branch, x_proposal, edge_index_pp, edge_index_bb, edge_index_bp,
           W_in_b, b_in_b, W_in_p, b_in_p,
           W1_pp, a1_pp_src, a1_pp_dst, bias1_pp,
           W1_bb, a1_bb_src, a1_bb_dst, bias1_bb,
           W1_bp_src, W1_bp_dst, a1_bp_src, a1_bp_dst, bias1_bp,
           W2_pp, a2_pp_src, a2_pp_dst, bias2_pp,
           W2_bp_src, W2_bp_dst, a2_bp_src, a2_bp_dst, bias2_bp,
           W_out, b_out):
    z1 = jnp.zeros((1,), jnp.float32)

    h_b = _linear(x_branch, W_in_b, b_in_b, slope=0.01)
    h_p = _linear(x_proposal, W_in_p, b_in_p, slope=0.01)

    def gat_shared(h, ei, W, a_s, a_d, rng):
        hs, as_v, ad_v = _gat_mm(h, W, a_s, a_d)
        num, s = _edge_phase(ei[0], ei[1], as_v, ad_v, hs, rng)
        return num, s, hs, as_v, ad_v

    def gat_bi(h_s, h_d, ei, W_s, W_d, a_s, a_d, rng):
        hs, as_v, _ = _gat_mm(h_s, W_s, a_s, a_s)
        wd = (W_d @ a_d).reshape(-1, 1)
        ad_v = _linear(h_d, wd, z1)[:, 0]
        num, s = _edge_phase(ei[0], ei[1], as_v, ad_v, hs, rng)
        return num, s

    nA, sA, hsA, asA, adA = gat_shared(h_p, edge_index_pp, W1_pp,
                                       a1_pp_src, a1_pp_dst, 1280)
    nB, sB = gat_bi(h_b, h_p, edge_index_bp, W1_bp_src, W1_bp_dst,
                    a1_bp_src, a1_bp_dst, 1280)
    p1 = _combine(nA, sA, hsA, asA, adA, bias1_pp, (nB, sB, bias1_bp), 0.01)

    nC, sC, hsC, asC, adC = gat_shared(h_b, edge_index_bb, W1_bb,
                                       a1_bb_src, a1_bb_dst, 1280)
    b1 = _combine(nC, sC, hsC, asC, adC, bias1_bb, None, 0.01)

    nD, sD, hsD, asD, adD = gat_shared(p1, edge_index_pp, W2_pp,
                                       a2_pp_src, a2_pp_dst, 2560)
    nE, sE = gat_bi(b1, p1, edge_index_bp, W2_bp_src, W2_bp_dst,
                    a2_bp_src, a2_bp_dst, 2560)
    p2 = _combine(nD, sD, hsD, asD, adD, bias2_pp, (nE, sE, bias2_bp), 0.01)

    return _linear(p2, W_out, b_out)


# speedup vs baseline: 1.0119x; 1.0119x over previous
import functools
import jax
import jax.numpy as jnp
from jax import lax
from jax.experimental import pallas as pl
from jax.experimental.pallas import tpu as pltpu
from jax.experimental.pallas import tpu_sc as plsc

_NC, _NS, _L = 2, 16, 16
_NW = _NC * _NS
_BLK = 1000


def _lin_body(x_ref, w_ref, b_ref, o_ref, *, slope):
    acc = jnp.dot(x_ref[...], w_ref[...], preferred_element_type=jnp.float32)
    acc = acc + b_ref[...]
    if slope is not None:
        acc = jnp.where(acc > 0, acc, slope * acc)
    o_ref[...] = acc


def _linear(x, W, b, slope=None, blk=_BLK):
    M, K = x.shape
    N = W.shape[1]
    return pl.pallas_call(
        functools.partial(_lin_body, slope=slope),
        grid=(M // blk,),
        in_specs=[
            pl.BlockSpec((blk, K), lambda i: (i, 0)),
            pl.BlockSpec((K, N), lambda i: (0, 0)),
            pl.BlockSpec((1, N), lambda i: (0, 0)),
        ],
        out_specs=pl.BlockSpec((blk, N), lambda i: (i, 0)),
        out_shape=jax.ShapeDtypeStruct((M, N), jnp.float32),
    )(x, W, b.reshape(1, N))


def _gat_mm_body(x_ref, w_ref, a2_ref, hs_ref, aa_ref):
    acc = jnp.dot(x_ref[...], w_ref[...], preferred_element_type=jnp.float32)
    hs_ref[...] = acc
    aa_ref[...] = jnp.dot(acc, a2_ref[...], preferred_element_type=jnp.float32)


def _gat_mm(x, W, a_src, a_dst, blk=_BLK):
    M, K = x.shape
    N = W.shape[1]
    A2 = jnp.stack([a_src, a_dst], axis=1)
    hs, aa = pl.pallas_call(
        _gat_mm_body,
        grid=(M // blk,),
        in_specs=[
            pl.BlockSpec((blk, K), lambda i: (i, 0)),
            pl.BlockSpec((K, N), lambda i: (0, 0)),
            pl.BlockSpec((N, 2), lambda i: (0, 0)),
        ],
        out_specs=[
            pl.BlockSpec((blk, N), lambda i: (i, 0)),
            pl.BlockSpec((blk, 2), lambda i: (i, 0)),
        ],
        out_shape=[
            jax.ShapeDtypeStruct((M, N), jnp.float32),
            jax.ShapeDtypeStruct((M, 2), jnp.float32),
        ],
    )(x, W, A2)
    return hs, aa[:, 0], aa[:, 1]


def _comb_body(*refs, two, slope):
    if two:
        (numA, sA, hs, aa, mA, biasA, numB, sB, biasB, o_ref) = refs
    else:
        (numA, sA, hs, aa, mA, biasA, o_ref) = refs
    aav = aa[...]
    ps = aav[:, 0:1] + aav[:, 1:2]
    ps = jnp.where(ps > 0, ps, 0.2 * ps)
    ps = jnp.exp(ps - mA[0, 0])
    nA = numA[0] + numA[1] + ps * hs[...]
    dA = sA[0] + sA[1] + ps + 1e-16
    o = nA / dA + biasA[...]
    if two:
        nB = numB[0] + numB[1]
        dB = sB[0] + sB[1] + 1e-16
        o = o + nB / dB + biasB[...]
    o_ref[...] = jnp.where(o > 0, o, slope * o)


def _combine(numA, sA, hs, as_A, ad_A, biasA, extraB, slope, blk=_BLK):
    M, HC = hs.shape
    npad = numA.shape[1]
    mA = jnp.maximum(jnp.max(as_A) + jnp.max(ad_A), 0.0).reshape(1, 1)
    aa = jnp.stack([as_A, ad_A], axis=1)
    two = extraB is not None
    ins = [numA, sA.reshape(_NC, npad, 1), hs, aa, mA, biasA.reshape(1, HC)]
    specs = [
        pl.BlockSpec((_NC, blk, HC), lambda i: (0, i, 0)),
        pl.BlockSpec((_NC, blk, 1), lambda i: (0, i, 0)),
        pl.BlockSpec((blk, HC), lambda i: (i, 0)),
        pl.BlockSpec((blk, 2), lambda i: (i, 0)),
        pl.BlockSpec((1, 1), lambda i: (0, 0)),
        pl.BlockSpec((1, HC), lambda i: (0, 0)),
    ]
    if two:
        numB, sB, biasB = extraB
        ins += [numB, sB.reshape(_NC, npad, 1), biasB.reshape(1, HC)]
        specs += [
            pl.BlockSpec((_NC, blk, HC), lambda i: (0, i, 0)),
            pl.BlockSpec((_NC, blk, 1), lambda i: (0, i, 0)),
            pl.BlockSpec((1, HC), lambda i: (0, 0)),
        ]
    return pl.pallas_call(
        functools.partial(_comb_body, two=two, slope=slope),
        grid=(M // blk,),
        in_specs=specs,
        out_specs=pl.BlockSpec((blk, HC), lambda i: (i, 0)),
        out_shape=jax.ShapeDtypeStruct((M, HC), jnp.float32),
    )(*ins)


def _edge_phase(src, dst, as_v, ad_v, hs, rng):
    E = src.shape[0]
    N, HC = hs.shape
    SL = 128
    nslab = HC // SL
    hs_parts = [hs[:, c * SL:(c + 1) * SL] for c in range(nslab)]
    assert E % _NW == 0
    ew = E // _NW
    assert ew % 8 == 0 and N % _L == 0
    nit = (ew + _L - 1) // _L
    nrounds = -(-N // rng)
    while (nrounds * rng) % (128 * _NS) != 0:
        nrounds += 1
    assert nrounds <= _L
    npad = nrounds * rng
    cap = max(nit * _L, -(-ew // 128) * 128)
    alloc = cap + _L
    nch = cap // 128
    rsub = rng // _NS
    nsub = npad // _NS
    G = 64

    def iota():
        return lax.iota(jnp.int32, _L)

    def body(src_h, dst_h, asv_h, adv_h, *rest):
        hs_hs = rest[:nslab]
        (z2_h, z1_h, num_h, s_h,
         asv, adv, srcb, dstb, p_orig, src_l, dloc_l, p_l,
         gidx0, sidx0, schunk_i, schunk_p) = rest[nslab:nslab + 16]
        k = nslab + 16
        rows0 = rest[k:k + nslab]
        accs = rest[k + nslab:k + 2 * nslab]
        (s_acc, sem, sem_g0, sem_s0) = rest[k + 2 * nslab:]
        c = lax.axis_index("c")
        s_ = lax.axis_index("s")
        wid = c * _NS + s_

        pltpu.sync_copy(asv_h, asv)
        pltpu.sync_copy(adv_h, adv)
        pltpu.sync_copy(src_h.at[pl.ds(wid * ew, ew)], srcb.at[pl.ds(0, ew)])
        pltpu.sync_copy(dst_h.at[pl.ds(wid * ew, ew)], dstb.at[pl.ds(0, ew)])

        zi = jnp.zeros((_L,), jnp.int32)
        zf = jnp.zeros((_L,), jnp.float32)
        j = ew
        while j < alloc - _L + 1:
            plsc.store_scatter(srcb, [j + iota()], zi)
            plsc.store_scatter(dstb, [j + iota()], zi)
            plsc.store_scatter(p_orig, [j + iota()], zf)
            plsc.store_scatter(src_l, [j + iota()], zi)
            plsc.store_scatter(dloc_l, [j + iota()], zi)
            j += _L

        def mbody(ref):
            def f(i, m):
                return jnp.maximum(m, plsc.load_gather(ref, [i * _L + iota()]))
            return jnp.max(lax.fori_loop(0, N // _L, f,
                                         jnp.full((_L,), -3e38, jnp.float32)))
        M = jnp.maximum(mbody(asv) + mbody(adv), 0.0)

        def cbody(i, car):
            pos = i * _L + iota()
            lm = pos < ew
            d16 = plsc.load_gather(dstb, [pos])
            rid = jnp.zeros((_L,), jnp.int32)
            for r in range(1, nrounds):
                rid = rid + (d16 >= r * rng).astype(jnp.int32)
            out = []
            for r in range(nrounds):
                m = (rid == r) & lm
                out.append(car[r] + plsc.all_reduce_population_count(m))
            return tuple(out)
        cnt = lax.fori_loop(0, nit, cbody,
                            tuple(jnp.zeros((_L,), jnp.int32)
                                  for _ in range(nrounds)))
        offv = []
        o = jnp.zeros((_L,), jnp.int32)
        for r in range(nrounds):
            offv.append(o)
            o = o + cnt[r]

        def pbody(i, car):
            pos = i * _L + iota()
            lm = pos < ew
            s16 = plsc.load_gather(srcb, [pos])
            d16 = plsc.load_gather(dstb, [pos])
            a = plsc.load_gather(asv, [s16]) + plsc.load_gather(adv, [d16])
            a = jnp.where(a > 0, a, 0.2 * a)
            p = jnp.exp(a - M)
            p = jnp.where(lm, p, 0.0)
            plsc.store_scatter(p_orig, [pos], p)
            rid = jnp.zeros((_L,), jnp.int32)
            for r in range(1, nrounds):
                rid = rid + (d16 >= r * rng).astype(jnp.int32)
            dloc = d16 - rid * rng
            out = []
            for r in range(nrounds):
                m = (rid == r) & lm
                cs = plsc.cumsum(m.astype(jnp.int32))
                posr = jnp.maximum(offv[r] + car[r] + cs - 1, 0)
                plsc.store_scatter(src_l, [posr], s16, mask=m)
                plsc.store_scatter(dloc_l, [posr], dloc, mask=m)
                plsc.store_scatter(p_l, [posr], p, mask=m)
                out.append(car[r] + plsc.all_reduce_population_count(m))
            return tuple(out)
        lax.fori_loop(0, nit, pbody,
                      tuple(jnp.zeros((_L,), jnp.int32) for _ in range(nrounds)))

        n_vec = jnp.zeros((_L,), jnp.int32)
        off_vec = jnp.zeros((_L,), jnp.int32)
        for r in range(nrounds):
            lane = (iota() == r).astype(jnp.int32)
            n_vec = n_vec + cnt[r] * lane
            off_vec = off_vec + offv[r] * lane

        pltpu.sync_copy(z1_h, s_acc.at[pl.ds(s_ * nsub, nsub)])
        plsc.subcore_barrier()

        def sbody(ci, _):
            for j4 in range(8):
                ix = ci * 128 + j4 * _L + iota()
                schunk_i[pl.ds(j4 * _L, _L)] = plsc.load_gather(dstb, [ix])
                schunk_p[pl.ds(j4 * _L, _L)] = plsc.load_gather(p_orig, [ix])
            pltpu.sync_copy(schunk_p, s_acc.at[schunk_i], add=True)
            return 0
        lax.fori_loop(0, nch, sbody, 0)
        plsc.subcore_barrier()
        pltpu.sync_copy(s_acc.at[pl.ds(s_ * nsub, nsub)],
                        s_h.at[pl.ds(c * npad + s_ * nsub, nsub)])
        plsc.subcore_barrier()

        def round_body(r, _r):
            for cc in range(nslab):
                pltpu.sync_copy(z2_h.at[pl.ds(0, rsub)],
                                accs[cc].at[pl.ds(s_ * rsub, rsub)])
            plsc.subcore_barrier()

            lane = iota() == r
            n_r = jnp.max(jnp.where(lane, n_vec, 0))
            off_r = jnp.max(jnp.where(lane, off_vec, 0))
            buf = [(gidx0, sidx0, rows0, sem_g0, sem_s0)]

            def stage(base, gi, si):
                start = jnp.minimum(off_r + base, alloc - G)
                for j4 in range(G // _L):
                    ix = start + j4 * _L + iota()
                    gi[pl.ds(j4 * _L, _L)] = plsc.load_gather(src_l, [ix])
                    si[pl.ds(j4 * _L, _L)] = plsc.load_gather(dloc_l, [ix])

            def fire_g(which):
                gi, _si, rws, sg, _ss = buf[which]
                for cc in range(nslab):
                    pltpu.async_copy(hs_hs[cc].at[gi], rws[cc], sg)

            def wait_g(which):
                gi, _si, rws, sg, _ss = buf[which]
                for cc in range(nslab):
                    pltpu.make_async_copy(hs_hs[cc].at[gi], rws[cc], sg).wait()

            def fire_s(which):
                _gi, si, rws, _sg, ss = buf[which]
                for cc in range(nslab):
                    pltpu.async_copy(rws[cc], accs[cc].at[si], ss, add=True)

            def wait_s(which):
                _gi, si, rws, _sg, ss = buf[which]
                for cc in range(nslab):
                    pltpu.make_async_copy(rws[cc], accs[cc].at[si], ss).wait()

            def scale(which, base):
                _gi, _si, rws, _sg, _ss = buf[which]

                def gb(g, _2):
                    for j in range(_L):
                        ro = g * _L + j
                        pv = plsc.load_gather(
                            p_l, [jnp.broadcast_to(off_r + base + ro, (_L,))])
                        valid = (base + ro < n_r).astype(jnp.float32)
                        pv = pv * valid
                        ridx = jnp.broadcast_to(ro, (_L,))
                        for cc in range(nslab):
                            for c4 in range(SL // _L):
                                cidx = c4 * _L + iota()
                                v = plsc.load_gather(rws[cc], [ridx, cidx])
                                plsc.store_scatter(rws[cc], [ridx, cidx],
                                                   v * pv)
                    return 0
                lax.fori_loop(0, G // _L, gb, 0)

            def bbody(b, _):
                base = b * G
                stage(base, gidx0, sidx0)
                fire_g(0)
                wait_g(0)
                scale(0, base)
                fire_s(0)
                wait_s(0)
                return 0
            nb = lax.shift_right_logical(n_r + (G - 1), G.bit_length() - 1)
            lax.fori_loop(0, nb, bbody, 0)
            plsc.subcore_barrier()

            for cc in range(nslab):
                pltpu.sync_copy(
                    accs[cc].at[pl.ds(s_ * rsub, rsub)],
                    num_h.at[pl.ds(c * npad + r * rng + s_ * rsub, rsub),
                             pl.ds(cc * SL, SL)])
            plsc.subcore_barrier()
            return 0
        lax.fori_loop(0, nrounds, round_body, 0)

    z2 = jnp.zeros((max(rsub, G), SL), jnp.float32)
    z1 = jnp.zeros((nsub,), jnp.float32)
    mesh = plsc.VectorSubcoreMesh(core_axis_name="c", subcore_axis_name="s")
    f = pl.kernel(
        body,
        out_type=[
            jax.ShapeDtypeStruct((_NC * npad, HC), jnp.float32),
            jax.ShapeDtypeStruct((_NC * npad,), jnp.float32),
        ],
        mesh=mesh,
        compiler_params=pltpu.CompilerParams(needs_layout_passes=False),
        scratch_types=[
            pltpu.VMEM((N,), jnp.float32),
            pltpu.VMEM((N,), jnp.float32),
            pltpu.VMEM((alloc,), jnp.int32),
            pltpu.VMEM((alloc,), jnp.int32),
            pltpu.VMEM((alloc,), jnp.float32),
            pltpu.VMEM((alloc,), jnp.int32),
            pltpu.VMEM((alloc,), jnp.int32),
            pltpu.VMEM((alloc,), jnp.float32),
            pltpu.VMEM((G,), jnp.int32),
            pltpu.VMEM((G,), jnp.int32),
            pltpu.VMEM((128,), jnp.int32),
            pltpu.VMEM((128,), jnp.float32),
        ] + [pltpu.VMEM((G, SL), jnp.float32) for _ in range(nslab)]
        + [pltpu.VMEM_SHARED((rng, SL), jnp.float32) for _ in range(nslab)]
        + [
            pltpu.VMEM_SHARED((npad,), jnp.float32),
            pltpu.SemaphoreType.DMA,
            pltpu.SemaphoreType.DMA,
            pltpu.SemaphoreType.DMA,
        ],
    )
    num, s = f(src, dst, as_v, ad_v, *hs_parts, z2, z1)
    return num.reshape(_NC, npad, HC), s.reshape(_NC, npad)


def kernel(x_branch, x_proposal, edge_index_pp, edge_index_bb, edge_index_bp,
           W_in_b, b_in_b, W_in_p, b_in_p,
           W1_pp, a1_pp_src, a1_pp_dst, bias1_pp,
           W1_bb, a1_bb_src, a1_bb_dst, bias1_bb,
           W1_bp_src, W1_bp_dst, a1_bp_src, a1_bp_dst, bias1_bp,
           W2_pp, a2_pp_src, a2_pp_dst, bias2_pp,
           W2_bp_src, W2_bp_dst, a2_bp_src, a2_bp_dst, bias2_bp,
           W_out, b_out):
    z1 = jnp.zeros((1,), jnp.float32)

    h_b = _linear(x_branch, W_in_b, b_in_b, slope=0.01)
    h_p = _linear(x_proposal, W_in_p, b_in_p, slope=0.01)

    def gat_shared(h, ei, W, a_s, a_d, rng):
        hs, as_v, ad_v = _gat_mm(h, W, a_s, a_d)
        num, s = _edge_phase(ei[0], ei[1], as_v, ad_v, hs, rng)
        return num, s, hs, as_v, ad_v

    def gat_bi(h_s, h_d, ei, W_s, W_d, a_s, a_d, rng):
        hs, as_v, _ = _gat_mm(h_s, W_s, a_s, a_s)
        wd = (W_d @ a_d).reshape(-1, 1)
        ad_v = _linear(h_d, wd, z1)[:, 0]
        num, s = _edge_phase(ei[0], ei[1], as_v, ad_v, hs, rng)
        return num, s

    nA, sA, hsA, asA, adA = gat_shared(h_p, edge_index_pp, W1_pp,
                                       a1_pp_src, a1_pp_dst, 1280)
    nB, sB = gat_bi(h_b, h_p, edge_index_bp, W1_bp_src, W1_bp_dst,
                    a1_bp_src, a1_bp_dst, 1280)
    p1 = _combine(nA, sA, hsA, asA, adA, bias1_pp, (nB, sB, bias1_bp), 0.01)

    nC, sC, hsC, asC, adC = gat_shared(h_b, edge_index_bb, W1_bb,
                                       a1_bb_src, a1_bb_dst, 1280)
    b1 = _combine(nC, sC, hsC, asC, adC, bias1_bb, None, 0.01)

    nD, sD, hsD, asD, adD = gat_shared(p1, edge_index_pp, W2_pp,
                                       a2_pp_src, a2_pp_dst, 2560)
    nE, sE = gat_bi(b1, p1, edge_index_bp, W2_bp_src, W2_bp_dst,
                    a2_bp_src, a2_bp_dst, 2560)
    p2 = _combine(nD, sD, hsD, asD, adD, bias2_pp, (nE, sE, bias2_bp), 0.01)

    return _linear(p2, W_out, b_out)
